# Initial kernel scaffold; baseline (speedup 1.0000x reference)
import jax
import jax.numpy as jnp
from jax.experimental import pallas as pl


def kernel(x, edge_index, W_enc, b_enc, W1, W2, W3, W_dec, b_dec):
    raise NotImplementedError("write your pallas kernel here")


# speedup vs baseline: 6.9698x; 6.9698x over previous
import functools

import jax
import jax.numpy as jnp
from jax import lax
from jax.experimental import pallas as pl
from jax.experimental.pallas import tpu as pltpu
from jax.experimental.pallas import tpu_sc as plsc

N = 10000
D = 128
E = 320000

NC = 2
NS = 16
NW = NC * NS

K = 128
CH = 80
CHH = 40
EPAD = NW * CH * K
RPT = 632
TPAD = NS * RPT

BLK = 632
GRID = TPAD // BLK

@functools.lru_cache(maxsize=1)
def _mesh():
    return plsc.VectorSubcoreMesh(
        core_axis_name="c", subcore_axis_name="s", num_cores=NC, num_subcores=NS
    )


def _hist_body(dst_hbm, ones_hbm, zeros_hbm, cnt_hbm, dstb, onesb, cnt):
    cid = lax.axis_index("c")
    sid = lax.axis_index("s")
    wid = sid * NC + cid
    rsl = pl.ds(sid * RPT, RPT)
    pltpu.sync_copy(zeros_hbm.at[rsl], cnt.at[rsl])
    pltpu.sync_copy(ones_hbm, onesb)
    pltpu.sync_copy(dst_hbm.at[wid], dstb)
    plsc.subcore_barrier()

    @pl.loop(0, CH)
    def _(j):
        pltpu.sync_copy(onesb, cnt.at[dstb.at[j]], add=True)

    plsc.subcore_barrier()
    pltpu.sync_copy(cnt.at[rsl], cnt_hbm.at[cid].at[rsl])


@jax.jit
def _sc_hist(dst3, ones16, zeros16):
    kern = pl.kernel(
        _hist_body,
        out_type=jax.ShapeDtypeStruct((NC, TPAD, 16), jnp.float32),
        mesh=_mesh(),
        scratch_types=[
            pltpu.VMEM((CH, K), jnp.int32),
            pltpu.VMEM((K, 16), jnp.float32),
            pltpu.VMEM_SHARED((TPAD, 16), jnp.float32),
        ],
    )
    return kern(dst3, ones16, zeros16)


def _layer_sc_body(g_hbm, z_hbm, src_hbm, dst_hbm, out_hbm,
                   srcb, dstb, rows0, rows1, sem0, sem1, acc):
    cid = lax.axis_index("c")
    sid = lax.axis_index("s")
    wid = sid * NC + cid
    rsl = pl.ds(sid * RPT, RPT)

    @pl.when(cid == 0)
    def _():
        pltpu.sync_copy(g_hbm.at[rsl], acc.at[rsl])

    @pl.when(cid == 1)
    def _():
        pltpu.sync_copy(z_hbm.at[rsl], acc.at[rsl])

    plsc.subcore_barrier()

    for half in range(CH // CHH):
        pltpu.sync_copy(src_hbm.at[wid, pl.ds(half * CHH, CHH)], srcb)
        pltpu.sync_copy(dst_hbm.at[wid, pl.ds(half * CHH, CHH)], dstb)
        pltpu.async_copy(g_hbm.at[srcb.at[0]], rows0, sem0)

        @pl.loop(0, CHH, step=2)
        def _(j):
            pltpu.make_async_copy(g_hbm.at[srcb.at[0]], rows0, sem0).wait()
            pltpu.async_copy(g_hbm.at[srcb.at[j + 1]], rows1, sem1)
            pltpu.sync_copy(rows0, acc.at[dstb.at[j]], add=True)
            pltpu.make_async_copy(g_hbm.at[srcb.at[0]], rows1, sem1).wait()

            @pl.when(j + 2 < CHH)
            def _():
                pltpu.async_copy(g_hbm.at[srcb.at[j + 2]], rows0, sem0)

            pltpu.sync_copy(rows1, acc.at[dstb.at[j + 1]], add=True)

    plsc.subcore_barrier()
    pltpu.sync_copy(acc.at[rsl], out_hbm.at[cid].at[rsl])


@jax.jit
def _sc_layer(g, zeros128, src3, dst3):
    kern = pl.kernel(
        _layer_sc_body,
        out_type=jax.ShapeDtypeStruct((NC, TPAD, D), jnp.float32),
        mesh=_mesh(),
        scratch_types=[
            pltpu.VMEM((CHH, K), jnp.int32),
            pltpu.VMEM((CHH, K), jnp.int32),
            pltpu.VMEM((K, D), jnp.float32),
            pltpu.VMEM((K, D), jnp.float32),
            pltpu.SemaphoreType.DMA,
            pltpu.SemaphoreType.DMA,
            pltpu.VMEM_SHARED((TPAD, D), jnp.float32),
        ],
    )
    return kern(g, zeros128, src3, dst3)


def _enc_body(x_ref, w_ref, b_ref, c0_ref, c1_ref, g_ref, dinv_ref):
    i = pl.program_id(0)
    deg = c0_ref[0, :, 0:1] + c1_ref[0, :, 0:1] + 1.0
    dinv = lax.rsqrt(deg)
    rows = jax.lax.broadcasted_iota(jnp.int32, (BLK, 1), 0) + i * BLK
    dinv = jnp.where(rows < N, dinv, 0.0)
    x0 = jnp.dot(x_ref[...], w_ref[...], preferred_element_type=jnp.float32)
    x0 = x0 + b_ref[...]
    g_ref[...] = jnp.where(rows < N, x0 * dinv, 0.0)
    dinv_ref[...] = jnp.broadcast_to(dinv, (BLK, 16))


@jax.jit
def _tc_encode(x, w_enc, b_enc2, cnt):
    return pl.pallas_call(
        _enc_body,
        grid=(GRID,),
        in_specs=[
            pl.BlockSpec((BLK, D), lambda i: (i, 0)),
            pl.BlockSpec((D, D), lambda i: (0, 0)),
            pl.BlockSpec((1, D), lambda i: (0, 0)),
            pl.BlockSpec((1, BLK, 16), lambda i: (0, i, 0)),
            pl.BlockSpec((1, BLK, 16), lambda i: (1, i, 0)),
        ],
        out_specs=[
            pl.BlockSpec((BLK, D), lambda i: (i, 0)),
            pl.BlockSpec((BLK, 16), lambda i: (i, 0)),
        ],
        out_shape=[
            jax.ShapeDtypeStruct((TPAD, D), jnp.float32),
            jax.ShapeDtypeStruct((TPAD, 16), jnp.float32),
        ],
    )(x, w_enc, b_enc2, cnt, cnt)


def _mid_body(p0_ref, p1_ref, dinv_ref, w_ref, gout_ref):
    dinv = dinv_ref[:, 0:1]
    a = (p0_ref[0] + p1_ref[0]) * dinv
    t = jnp.dot(a, w_ref[...], preferred_element_type=jnp.float32)
    gout_ref[...] = jnp.maximum(t, 0.0) * dinv


@jax.jit
def _tc_mid(part, dinv, w):
    return pl.pallas_call(
        _mid_body,
        grid=(GRID,),
        in_specs=[
            pl.BlockSpec((1, BLK, D), lambda i: (0, i, 0)),
            pl.BlockSpec((1, BLK, D), lambda i: (1, i, 0)),
            pl.BlockSpec((BLK, 16), lambda i: (i, 0)),
            pl.BlockSpec((D, D), lambda i: (0, 0)),
        ],
        out_specs=pl.BlockSpec((BLK, D), lambda i: (i, 0)),
        out_shape=jax.ShapeDtypeStruct((TPAD, D), jnp.float32),
    )(part, part, dinv, w)


def _final_body(p0_ref, p1_ref, dinv_ref, w_ref, wd_ref, bd_ref, out_ref):
    dinv = dinv_ref[:, 0:1]
    a = (p0_ref[0] + p1_ref[0]) * dinv
    h = jnp.maximum(
        jnp.dot(a, w_ref[...], preferred_element_type=jnp.float32), 0.0
    )
    out_ref[...] = (
        jnp.dot(h, wd_ref[...], preferred_element_type=jnp.float32)
        + bd_ref[...]
    )


@jax.jit
def _tc_final(part, dinv, w3, w_dec, b_dec2):
    return pl.pallas_call(
        _final_body,
        grid=(GRID,),
        in_specs=[
            pl.BlockSpec((1, BLK, D), lambda i: (0, i, 0)),
            pl.BlockSpec((1, BLK, D), lambda i: (1, i, 0)),
            pl.BlockSpec((BLK, 16), lambda i: (i, 0)),
            pl.BlockSpec((D, D), lambda i: (0, 0)),
            pl.BlockSpec((D, 1), lambda i: (0, 0)),
            pl.BlockSpec((1, 1), lambda i: (0, 0)),
        ],
        out_specs=pl.BlockSpec((BLK, 1), lambda i: (i, 0)),
        out_shape=jax.ShapeDtypeStruct((N, 1), jnp.float32),
    )(part, part, dinv, w3, w_dec, b_dec2)


def kernel(x, edge_index, W_enc, b_enc, W1, W2, W3, W_dec, b_dec):
    ei = edge_index.astype(jnp.int32)
    src = jnp.concatenate([ei[0], jnp.zeros((EPAD - E,), jnp.int32)])
    dst = jnp.concatenate(
        [ei[1], jnp.full((EPAD - E,), N, jnp.int32)]
    )
    src3 = src.reshape(NW, CH, K)
    dst3 = dst.reshape(NW, CH, K)

    ones16 = jnp.ones((K, 16), jnp.float32)
    zeros16 = jnp.zeros((TPAD, 16), jnp.float32)
    zeros128 = jnp.zeros((TPAD, D), jnp.float32)

    cnt = _sc_hist(dst3, ones16, zeros16)
    g, dinv = _tc_encode(x, W_enc, b_enc.reshape(1, D), cnt)
    for w in (W1, W2):
        part = _sc_layer(g, zeros128, src3, dst3)
        g = _tc_mid(part, dinv, w)
    part = _sc_layer(g, zeros128, src3, dst3)
    return _tc_final(part, dinv, W3, W_dec, b_dec.reshape(1, 1))


# speedup vs baseline: 22.9052x; 3.2863x over previous
import functools

import jax
import jax.numpy as jnp
from jax import lax
from jax.experimental import pallas as pl
from jax.experimental.pallas import tpu as pltpu
from jax.experimental.pallas import tpu_sc as plsc

N = 10000
D = 128
E = 320000

NC = 2
NS = 16
NW = NC * NS

K = 128
CH = 80
CHH = 40
EPAD = NW * CH * K
RPT = 632
TPAD = NS * RPT

BLK = 632
GRID = TPAD // BLK

@functools.lru_cache(maxsize=1)
def _mesh():
    return plsc.VectorSubcoreMesh(
        core_axis_name="c", subcore_axis_name="s", num_cores=NC, num_subcores=NS
    )


def _hist_body(dst_hbm, ones_hbm, zeros_hbm, cnt_hbm, dstb, onesb, cnt):
    cid = lax.axis_index("c")
    sid = lax.axis_index("s")
    wid = sid * NC + cid
    rsl = pl.ds(sid * RPT, RPT)
    pltpu.sync_copy(zeros_hbm.at[rsl], cnt.at[rsl])
    pltpu.sync_copy(ones_hbm, onesb)
    pltpu.sync_copy(dst_hbm.at[wid], dstb)
    plsc.subcore_barrier()

    @pl.loop(0, CH)
    def _(j):
        pltpu.sync_copy(onesb, cnt.at[dstb.at[j]], add=True)

    plsc.subcore_barrier()
    pltpu.sync_copy(cnt.at[rsl], cnt_hbm.at[cid].at[rsl])


@jax.jit
def _sc_hist(dst3, ones16, zeros16):
    kern = pl.kernel(
        _hist_body,
        out_type=jax.ShapeDtypeStruct((NC, TPAD, 16), jnp.float32),
        mesh=_mesh(),
        scratch_types=[
            pltpu.VMEM((CH, K), jnp.int32),
            pltpu.VMEM((K, 16), jnp.float32),
            pltpu.VMEM_SHARED((TPAD, 16), jnp.float32),
        ],
    )
    return kern(dst3, ones16, zeros16)


def _layer_sc_body(g_hbm, z_hbm, src_hbm, dst_hbm, out_hbm,
                   srcb, dstb, rows0, rows1, sem0, sem1, acc):
    cid = lax.axis_index("c")
    sid = lax.axis_index("s")
    wid = sid * NC + cid
    rsl = pl.ds(sid * RPT, RPT)

    @pl.when(cid == 0)
    def _():
        pltpu.sync_copy(g_hbm.at[rsl], acc.at[rsl])

    @pl.when(cid == 1)
    def _():
        pltpu.sync_copy(z_hbm.at[rsl], acc.at[rsl])

    plsc.subcore_barrier()

    for half in range(CH // CHH):
        pltpu.sync_copy(src_hbm.at[wid, pl.ds(half * CHH, CHH)], srcb)
        pltpu.sync_copy(dst_hbm.at[wid, pl.ds(half * CHH, CHH)], dstb)
        pltpu.async_copy(g_hbm.at[srcb.at[0]], rows0, sem0)

        @pl.loop(0, CHH, step=2)
        def _(j):
            pltpu.make_async_copy(g_hbm.at[srcb.at[0]], rows0, sem0).wait()
            pltpu.async_copy(g_hbm.at[srcb.at[j + 1]], rows1, sem1)
            pltpu.sync_copy(rows0, acc.at[dstb.at[j]], add=True)
            pltpu.make_async_copy(g_hbm.at[srcb.at[0]], rows1, sem1).wait()

            @pl.when(j + 2 < CHH)
            def _():
                pltpu.async_copy(g_hbm.at[srcb.at[j + 2]], rows0, sem0)

            pltpu.sync_copy(rows1, acc.at[dstb.at[j + 1]], add=True)

    plsc.subcore_barrier()
    pltpu.sync_copy(acc.at[rsl], out_hbm.at[cid].at[rsl])


@jax.jit
def _sc_layer(g, zeros128, src3, dst3):
    kern = pl.kernel(
        _layer_sc_body,
        out_type=jax.ShapeDtypeStruct((NC, TPAD, D), jnp.float32),
        mesh=_mesh(),
        scratch_types=[
            pltpu.VMEM((CHH, K), jnp.int32),
            pltpu.VMEM((CHH, K), jnp.int32),
            pltpu.VMEM((K, D), jnp.float32),
            pltpu.VMEM((K, D), jnp.float32),
            pltpu.SemaphoreType.DMA,
            pltpu.SemaphoreType.DMA,
            pltpu.VMEM_SHARED((TPAD, D), jnp.float32),
        ],
    )
    return kern(g, zeros128, src3, dst3)


def _enc_body(x_ref, w_ref, b_ref, c0_ref, c1_ref, g_ref, dinv_ref):
    i = pl.program_id(0)
    deg = c0_ref[0, :, 0:1] + c1_ref[0, :, 0:1] + 1.0
    dinv = lax.rsqrt(deg)
    rows = jax.lax.broadcasted_iota(jnp.int32, (BLK, 1), 0) + i * BLK
    dinv = jnp.where(rows < N, dinv, 0.0)
    x0 = jnp.dot(x_ref[...], w_ref[...], preferred_element_type=jnp.float32)
    x0 = x0 + b_ref[...]
    g_ref[...] = jnp.where(rows < N, x0 * dinv, 0.0)
    dinv_ref[...] = jnp.broadcast_to(dinv, (BLK, 16))


@jax.jit
def _tc_encode(x, w_enc, b_enc2, cnt):
    return pl.pallas_call(
        _enc_body,
        grid=(GRID,),
        in_specs=[
            pl.BlockSpec((BLK, D), lambda i: (i, 0)),
            pl.BlockSpec((D, D), lambda i: (0, 0)),
            pl.BlockSpec((1, D), lambda i: (0, 0)),
            pl.BlockSpec((1, BLK, 16), lambda i: (0, i, 0)),
            pl.BlockSpec((1, BLK, 16), lambda i: (1, i, 0)),
        ],
        out_specs=[
            pl.BlockSpec((BLK, D), lambda i: (i, 0)),
            pl.BlockSpec((BLK, 16), lambda i: (i, 0)),
        ],
        out_shape=[
            jax.ShapeDtypeStruct((TPAD, D), jnp.float32),
            jax.ShapeDtypeStruct((TPAD, 16), jnp.float32),
        ],
    )(x, w_enc, b_enc2, cnt, cnt)


def _mid_body(p0_ref, p1_ref, dinv_ref, w_ref, gout_ref):
    dinv = dinv_ref[:, 0:1]
    a = (p0_ref[0] + p1_ref[0]) * dinv
    t = jnp.dot(a, w_ref[...], preferred_element_type=jnp.float32)
    gout_ref[...] = jnp.maximum(t, 0.0) * dinv


@jax.jit
def _tc_mid(part, dinv, w):
    return pl.pallas_call(
        _mid_body,
        grid=(GRID,),
        in_specs=[
            pl.BlockSpec((1, BLK, D), lambda i: (0, i, 0)),
            pl.BlockSpec((1, BLK, D), lambda i: (1, i, 0)),
            pl.BlockSpec((BLK, 16), lambda i: (i, 0)),
            pl.BlockSpec((D, D), lambda i: (0, 0)),
        ],
        out_specs=pl.BlockSpec((BLK, D), lambda i: (i, 0)),
        out_shape=jax.ShapeDtypeStruct((TPAD, D), jnp.float32),
    )(part, part, dinv, w)


def _final_body(p0_ref, p1_ref, dinv_ref, w_ref, wd_ref, bd_ref, out_ref):
    dinv = dinv_ref[:, 0:1]
    a = (p0_ref[0] + p1_ref[0]) * dinv
    h = jnp.maximum(
        jnp.dot(a, w_ref[...], preferred_element_type=jnp.float32), 0.0
    )
    out_ref[...] = (
        jnp.dot(h, wd_ref[...], preferred_element_type=jnp.float32)
        + bd_ref[...]
    )


@jax.jit
def _tc_final(part, dinv, w3, w_dec, b_dec2):
    return pl.pallas_call(
        _final_body,
        grid=(GRID,),
        in_specs=[
            pl.BlockSpec((1, BLK, D), lambda i: (0, i, 0)),
            pl.BlockSpec((1, BLK, D), lambda i: (1, i, 0)),
            pl.BlockSpec((BLK, 16), lambda i: (i, 0)),
            pl.BlockSpec((D, D), lambda i: (0, 0)),
            pl.BlockSpec((D, 1), lambda i: (0, 0)),
            pl.BlockSpec((1, 1), lambda i: (0, 0)),
        ],
        out_specs=pl.BlockSpec((BLK, 1), lambda i: (i, 0)),
        out_shape=jax.ShapeDtypeStruct((N, 1), jnp.float32),
    )(part, part, dinv, w3, w_dec, b_dec2)


def kernel(x, edge_index, W_enc, b_enc, W1, W2, W3, W_dec, b_dec):
    ei = edge_index.astype(jnp.int32)
    epw = E // NW
    pad = CH * K - epw
    psrc = jnp.broadcast_to((jnp.arange(pad) * 37) % N, (NW, pad))
    pdst = jnp.broadcast_to(N + jnp.arange(pad) % (TPAD - N), (NW, pad))
    src3 = jnp.concatenate(
        [ei[0].reshape(NW, epw), psrc.astype(jnp.int32)], axis=1
    ).reshape(NW, CH, K)
    dst3 = jnp.concatenate(
        [ei[1].reshape(NW, epw), pdst.astype(jnp.int32)], axis=1
    ).reshape(NW, CH, K)

    ones16 = jnp.ones((K, 16), jnp.float32)
    zeros16 = jnp.zeros((TPAD, 16), jnp.float32)
    zeros128 = jnp.zeros((TPAD, D), jnp.float32)

    cnt = _sc_hist(dst3, ones16, zeros16)
    g, dinv = _tc_encode(x, W_enc, b_enc.reshape(1, D), cnt)
    for w in (W1, W2):
        part = _sc_layer(g, zeros128, src3, dst3)
        g = _tc_mid(part, dinv, w)
    part = _sc_layer(g, zeros128, src3, dst3)
    return _tc_final(part, dinv, W3, W_dec, b_dec.reshape(1, 1))
